# Initial kernel scaffold; baseline (speedup 1.0000x reference)
import jax
import jax.numpy as jnp
from jax.experimental import pallas as pl


def kernel(x, embed):
    raise NotImplementedError("write your pallas kernel here")


# speedup vs baseline: 1.0152x; 1.0152x over previous
import functools

import jax
import jax.numpy as jnp
from jax import lax
from jax.experimental import pallas as pl
from jax.experimental.pallas import tpu as pltpu
from jax.experimental.pallas import tpu_sc as plsc

DIM = 256
INTERPRET = False


_BR = 256


def _tc_body(x_ref, e_ref, idx_ref, loss_ref):
    i = pl.program_id(0)
    ni = pl.num_programs(0)
    x = x_ref[...]
    e = e_ref[...]
    cb = e.shape[0]
    xe2 = lax.dot_general(
        x * 2.0, e,
        dimension_numbers=(((1,), (1,)), ((), ())),
        preferred_element_type=jnp.float32,
    )
    x2 = jnp.sum(x * x, axis=1, keepdims=True)
    e2 = jnp.sum(e * e, axis=1, keepdims=True).T
    dist = (xe2 - x2) - e2
    m = jnp.max(dist, axis=1, keepdims=True)
    iota = lax.broadcasted_iota(jnp.int32, dist.shape, 1)
    idx = jnp.min(jnp.where(dist == m, iota, cb), axis=1, keepdims=True)
    idx_ref[...] = idx
    contrib = jnp.sum(-m).reshape(1, 1)
    prev = jnp.where(i == 0, jnp.zeros((1, 1), jnp.float32), loss_ref[...])
    total = prev + contrib
    loss_ref[...] = jnp.where(i == ni - 1, total / (ni * _BR * DIM), total)


def _tc_dist_argmax(xf, embed):
    m, d = xf.shape
    cb = embed.shape[0]
    grid = (m // _BR,)
    return pl.pallas_call(
        _tc_body,
        grid=grid,
        in_specs=[
            pl.BlockSpec((_BR, d), lambda i: (i, 0)),
            pl.BlockSpec((cb, d), lambda i: (0, 0)),
        ],
        out_specs=[
            pl.BlockSpec((_BR, 1), lambda i: (i, 0)),
            pl.BlockSpec((1, 1), lambda i: (0, 0)),
        ],
        out_shape=[
            jax.ShapeDtypeStruct((m, 1), jnp.int32),
            jax.ShapeDtypeStruct((1, 1), jnp.float32),
        ],
        compiler_params=pltpu.CompilerParams(
            dimension_semantics=("arbitrary",),
        ),
        interpret=INTERPRET,
    )(xf, embed)


_CH = 128


def _sc_gather(table, idx2d):
    v, d = table.shape
    nrow, ch = idx2d.shape
    b = nrow * ch
    info = plsc.get_sparse_core_info()
    nw = info.num_cores * info.num_subcores
    nch = b // (nw * ch)
    mesh = plsc.VectorSubcoreMesh(core_axis_name="c", subcore_axis_name="s")

    @functools.partial(
        pl.kernel,
        mesh=mesh,
        out_type=jax.ShapeDtypeStruct((b, d), jnp.float32),
        scratch_types=[
            pltpu.VMEM((nch, ch), jnp.int32),
            pltpu.VMEM((ch, d), jnp.float32),
            pltpu.VMEM((ch, d), jnp.float32),
            pltpu.SemaphoreType.DMA,
            pltpu.SemaphoreType.DMA,
        ],
    )
    def k(table_hbm, idx_hbm, out_hbm, idx_v, buf0, buf1, sem0, sem1):
        wid = lax.axis_index("s") * info.num_cores + lax.axis_index("c")
        pltpu.sync_copy(idx_hbm.at[pl.ds(wid * nch, nch)], idx_v)
        bufs = (buf0, buf1)
        sems = (sem0, sem1)
        copies = [None, None]
        copies[0] = pltpu.async_copy(table_hbm.at[idx_v.at[0]], buf0, sem0)
        for c in range(nch):
            if c + 1 < nch:
                copies[(c + 1) % 2] = pltpu.async_copy(
                    table_hbm.at[idx_v.at[c + 1]], bufs[(c + 1) % 2],
                    sems[(c + 1) % 2])
            copies[c % 2].wait()
            pltpu.sync_copy(
                bufs[c % 2],
                out_hbm.at[pl.ds(wid * nch * ch + c * ch, ch)])

    return k(table, idx2d)


def kernel(x, embed):
    shape = x.shape
    xf = x.reshape(-1, shape[-1])
    idx2, loss = _tc_dist_argmax(xf, embed)
    idx = idx2.reshape(-1)
    q = _sc_gather(embed, idx.reshape(-1, _CH))
    return (q.reshape(shape), idx.reshape(shape[:-1]), loss.reshape(()))


# speedup vs baseline: 1.2197x; 1.2014x over previous
import functools

import jax
import jax.numpy as jnp
from jax import lax
from jax.experimental import pallas as pl
from jax.experimental.pallas import tpu as pltpu
from jax.experimental.pallas import tpu_sc as plsc

DIM = 256
INTERPRET = False


_BR = 256


def _tc_body(x_ref, e_ref, idx_ref, loss_ref, e2_ref):
    i = pl.program_id(0)
    ni = pl.num_programs(0)
    x = x_ref[...]
    e = e_ref[...]
    cb = e.shape[0]

    @pl.when(i == 0)
    def _():
        e2_ref[...] = jnp.sum(e * e, axis=1, keepdims=True).T
    xe2 = lax.dot_general(
        x * 2.0, e,
        dimension_numbers=(((1,), (1,)), ((), ())),
        preferred_element_type=jnp.float32,
    )
    x2 = jnp.sum(x * x, axis=1, keepdims=True)
    dist = (xe2 - x2) - e2_ref[...]
    m = jnp.max(dist, axis=1, keepdims=True)
    iota = lax.broadcasted_iota(jnp.int32, dist.shape, 1)
    idx = jnp.min(jnp.where(dist == m, iota, cb), axis=1, keepdims=True)
    idx_ref[...] = idx
    contrib = jnp.sum(-m).reshape(1, 1)
    prev = jnp.where(i == 0, jnp.zeros((1, 1), jnp.float32), loss_ref[...])
    total = prev + contrib
    loss_ref[...] = jnp.where(i == ni - 1, total / (ni * _BR * DIM), total)


def _tc_dist_argmax(xf, embed):
    m, d = xf.shape
    cb = embed.shape[0]
    grid = (m // _BR,)
    return pl.pallas_call(
        _tc_body,
        grid=grid,
        in_specs=[
            pl.BlockSpec((_BR, d), lambda i: (i, 0)),
            pl.BlockSpec((cb, d), lambda i: (0, 0)),
        ],
        out_specs=[
            pl.BlockSpec((_BR, 1), lambda i: (i, 0)),
            pl.BlockSpec((1, 1), lambda i: (0, 0)),
        ],
        out_shape=[
            jax.ShapeDtypeStruct((m, 1), jnp.int32),
            jax.ShapeDtypeStruct((1, 1), jnp.float32),
        ],
        scratch_shapes=[pltpu.VMEM((1, cb), jnp.float32)],
        compiler_params=pltpu.CompilerParams(
            dimension_semantics=("arbitrary",),
        ),
        interpret=INTERPRET,
    )(xf, embed)


_CH = 128


def _sc_gather(table, idx2d):
    v, d = table.shape
    nrow, ch = idx2d.shape
    b = nrow * ch
    info = plsc.get_sparse_core_info()
    nw = info.num_cores * info.num_subcores
    nch = b // (nw * ch)
    mesh = plsc.VectorSubcoreMesh(core_axis_name="c", subcore_axis_name="s")

    @functools.partial(
        pl.kernel,
        mesh=mesh,
        out_type=jax.ShapeDtypeStruct((b, d), jnp.float32),
        scratch_types=[
            pltpu.VMEM((nch, ch), jnp.int32),
            pltpu.VMEM((ch, d), jnp.float32),
            pltpu.VMEM((ch, d), jnp.float32),
            pltpu.SemaphoreType.DMA,
            pltpu.SemaphoreType.DMA,
        ],
    )
    def k(table_hbm, idx_hbm, out_hbm, idx_v, buf0, buf1, sem0, sem1):
        wid = lax.axis_index("s") * info.num_cores + lax.axis_index("c")
        pltpu.sync_copy(idx_hbm.at[pl.ds(wid * nch, nch)], idx_v)
        bufs = (buf0, buf1)
        sems = (sem0, sem1)
        copies = [None, None]
        copies[0] = pltpu.async_copy(table_hbm.at[idx_v.at[0]], buf0, sem0)
        for c in range(nch):
            if c + 1 < nch:
                copies[(c + 1) % 2] = pltpu.async_copy(
                    table_hbm.at[idx_v.at[c + 1]], bufs[(c + 1) % 2],
                    sems[(c + 1) % 2])
            copies[c % 2].wait()
            pltpu.sync_copy(
                bufs[c % 2],
                out_hbm.at[pl.ds(wid * nch * ch + c * ch, ch)])

    return k(table, idx2d)


def kernel(x, embed):
    shape = x.shape
    xf = x.reshape(-1, shape[-1])
    idx2, loss = _tc_dist_argmax(xf, embed)
    idx = idx2.reshape(-1)
    q = _sc_gather(embed, idx.reshape(-1, _CH))
    return (q.reshape(shape), idx.reshape(shape[:-1]), loss.reshape(()))


# speedup vs baseline: 1.3504x; 1.1071x over previous
import functools

import jax
import jax.numpy as jnp
from jax import lax
from jax.experimental import pallas as pl
from jax.experimental.pallas import tpu as pltpu
from jax.experimental.pallas import tpu_sc as plsc

DIM = 256
INTERPRET = False


_BR = 512


def _tc_body(x_ref, e_ref, idx_ref, loss_ref, e2_ref):
    i = pl.program_id(0)
    ni = pl.num_programs(0)
    x = x_ref[...]
    e = e_ref[...]
    cb = e.shape[0]

    @pl.when(i == 0)
    def _():
        e2_ref[...] = jnp.sum(e * e, axis=1, keepdims=True).T
    xe2 = lax.dot_general(
        x * 2.0, e,
        dimension_numbers=(((1,), (1,)), ((), ())),
        preferred_element_type=jnp.float32,
    )
    x2 = jnp.sum(x * x, axis=1, keepdims=True)
    score = xe2 - e2_ref[...]
    m = jnp.max(score, axis=1, keepdims=True)
    iota = lax.broadcasted_iota(jnp.int32, score.shape, 1)
    idx = jnp.min(jnp.where(score == m, iota, cb), axis=1, keepdims=True)
    idx_ref[...] = idx
    contrib = jnp.sum(x2 - m).reshape(1, 1)
    prev = jnp.where(i == 0, jnp.zeros((1, 1), jnp.float32), loss_ref[...])
    total = prev + contrib
    loss_ref[...] = jnp.where(i == ni - 1, total / (ni * _BR * DIM), total)


def _tc_dist_argmax(xf, embed):
    m, d = xf.shape
    cb = embed.shape[0]
    grid = (m // _BR,)
    return pl.pallas_call(
        _tc_body,
        grid=grid,
        in_specs=[
            pl.BlockSpec((_BR, d), lambda i: (i, 0)),
            pl.BlockSpec((cb, d), lambda i: (0, 0)),
        ],
        out_specs=[
            pl.BlockSpec((_BR, 1), lambda i: (i, 0)),
            pl.BlockSpec((1, 1), lambda i: (0, 0)),
        ],
        out_shape=[
            jax.ShapeDtypeStruct((m, 1), jnp.int32),
            jax.ShapeDtypeStruct((1, 1), jnp.float32),
        ],
        scratch_shapes=[pltpu.VMEM((1, cb), jnp.float32)],
        compiler_params=pltpu.CompilerParams(
            dimension_semantics=("arbitrary",),
        ),
        interpret=INTERPRET,
    )(xf, embed)


_CH = 128


def _sc_gather(table, idx2d):
    v, d = table.shape
    nrow, ch = idx2d.shape
    b = nrow * ch
    info = plsc.get_sparse_core_info()
    nw = info.num_cores * info.num_subcores
    nch = b // (nw * ch)
    mesh = plsc.VectorSubcoreMesh(core_axis_name="c", subcore_axis_name="s")

    @functools.partial(
        pl.kernel,
        mesh=mesh,
        out_type=jax.ShapeDtypeStruct((b, d), jnp.float32),
        scratch_types=[
            pltpu.VMEM((nch, ch), jnp.int32),
            pltpu.VMEM((ch, d), jnp.float32),
            pltpu.VMEM((ch, d), jnp.float32),
            pltpu.SemaphoreType.DMA,
            pltpu.SemaphoreType.DMA,
        ],
    )
    def k(table_hbm, idx_hbm, out_hbm, idx_v, buf0, buf1, sem0, sem1):
        wid = lax.axis_index("s") * info.num_cores + lax.axis_index("c")
        pltpu.sync_copy(idx_hbm.at[pl.ds(wid * nch, nch)], idx_v)
        bufs = (buf0, buf1)
        sems = (sem0, sem1)
        copies = [None, None]
        copies[0] = pltpu.async_copy(table_hbm.at[idx_v.at[0]], buf0, sem0)
        for c in range(nch):
            if c + 1 < nch:
                copies[(c + 1) % 2] = pltpu.async_copy(
                    table_hbm.at[idx_v.at[c + 1]], bufs[(c + 1) % 2],
                    sems[(c + 1) % 2])
            copies[c % 2].wait()
            pltpu.sync_copy(
                bufs[c % 2],
                out_hbm.at[pl.ds(wid * nch * ch + c * ch, ch)])

    return k(table, idx2d)


def kernel(x, embed):
    shape = x.shape
    xf = x.reshape(-1, shape[-1])
    idx2, loss = _tc_dist_argmax(xf, embed)
    idx = idx2.reshape(-1)
    q = _sc_gather(embed, idx.reshape(-1, _CH))
    return (q.reshape(shape), idx.reshape(shape[:-1]), loss.reshape(()))
